# baseline (speedup 1.0000x reference)
import functools

import jax
import jax.numpy as jnp
from jax.experimental import pallas as pl
from jax.experimental.pallas import tpu as pltpu

_EPS = 1e-12


def _ffn_body(x_ref, g1c_ref, b1_ref, wd_ref, bd_ref, g2_ref, b2_ref,
              mem_ref, wk_ref, bk_ref, wv_ref, bv_ref, g3c_ref, b3_ref,
              wu_ref, bu_ref, o_ref,
              wd_aug_s, wu_aug_s, key_s, val_s, const_s):
    H = x_ref.shape[1]
    D = wd_ref.shape[1]

    @pl.when(pl.program_id(0) == 0)
    def _prep():
        wdg = wd_ref[...] * g1c_ref[...]
        wd_aug_s[...] = jnp.concatenate(
            [wdg, jnp.ones((H, 1), jnp.float32)], axis=1)

        wug = wu_ref[...] * g3c_ref[...]
        bias_row = jnp.dot(b3_ref[...], wu_ref[...],
                           preferred_element_type=jnp.float32) + bu_ref[...]
        wu_aug_s[...] = jnp.concatenate([wug, bias_row], axis=0)

        mem = mem_ref[...]
        key_s[...] = jnp.dot(mem, wk_ref[...],
                             preferred_element_type=jnp.float32) + bk_ref[...]
        val_s[...] = jnp.dot(mem, wv_ref[...],
                             preferred_element_type=jnp.float32) + bv_ref[...]

        csum = jnp.sum(wdg, axis=0, keepdims=True)
        cb = jnp.dot(b1_ref[...], wd_ref[...],
                     preferred_element_type=jnp.float32) + bd_ref[...]
        const_s[0:1, :D] = csum
        const_s[1:2, :D] = cb

    x = x_ref[...]

    raw = jnp.dot(x, wd_aug_s[...], preferred_element_type=jnp.float32)
    xw = raw[:, :D]
    m = raw[:, D:D + 1] * (1.0 / H)

    sqsum = jnp.sum(x * x, axis=-1, keepdims=True)
    v = sqsum * (1.0 / H) - m * m
    s = jax.lax.rsqrt(v + _EPS)

    csum = const_s[0:1, :D]
    cb = const_s[1:2, :D]
    d = s * (xw - m * csum) + cb

    m2 = jnp.mean(d, axis=-1, keepdims=True)
    dc = d - m2
    v2 = jnp.mean(dc * dc, axis=-1, keepdims=True)
    q = dc * jax.lax.rsqrt(v2 + _EPS) * g2_ref[...] + b2_ref[...]

    logits = jax.lax.dot_general(q, key_s[...], (((1,), (1,)), ((), ())),
                                 preferred_element_type=jnp.float32)
    logits = logits - jnp.max(logits, axis=-1, keepdims=True)
    e = jnp.exp(logits)
    p = e / jnp.sum(e, axis=-1, keepdims=True)
    mo = jnp.dot(p, val_s[...], preferred_element_type=jnp.float32)

    m3 = jnp.mean(mo, axis=-1, keepdims=True)
    mc = mo - m3
    v3 = jnp.mean(mc * mc, axis=-1, keepdims=True)
    z = mc * jax.lax.rsqrt(v3 + _EPS)
    z_aug = jnp.concatenate([z, jnp.ones((z.shape[0], 1), jnp.float32)], axis=1)

    o_ref[...] = jnp.dot(z_aug, wu_aug_s[...], preferred_element_type=jnp.float32)


@functools.partial(jax.jit, static_argnames=("block_rows", "interpret"))
def _run(x2d, g1, b1, W_down, b_down, g2, b2, memory, W_k, b_k, W_v, b_v,
         g3, b3, W_up, b_up, block_rows=1024, interpret=False):
    n, H = x2d.shape
    D = W_down.shape[1]
    M = memory.shape[0]
    grid = (n // block_rows,)

    def full(a):
        return pl.BlockSpec(a.shape, lambda i: (0,) * a.ndim)

    ins = (g1.reshape(-1, 1), b1.reshape(1, -1), W_down, b_down.reshape(1, -1),
           g2.reshape(1, -1), b2.reshape(1, -1), memory, W_k, b_k.reshape(1, -1),
           W_v, b_v.reshape(1, -1), g3.reshape(-1, 1), b3.reshape(1, -1),
           W_up, b_up.reshape(1, -1))

    return pl.pallas_call(
        _ffn_body,
        out_shape=jax.ShapeDtypeStruct((n, H), jnp.float32),
        grid=grid,
        in_specs=[pl.BlockSpec((block_rows, H), lambda i: (i, 0))]
                 + [full(a) for a in ins],
        out_specs=pl.BlockSpec((block_rows, H), lambda i: (i, 0)),
        scratch_shapes=[
            pltpu.VMEM((H, D + 1), jnp.float32),
            pltpu.VMEM((D + 1, H), jnp.float32),
            pltpu.VMEM((M, D), jnp.float32),
            pltpu.VMEM((M, D), jnp.float32),
            pltpu.VMEM((8, 128), jnp.float32),
        ],
        compiler_params=pltpu.CompilerParams(
            dimension_semantics=("arbitrary",),
            vmem_limit_bytes=50 * 1024 * 1024,
        ),
        name="bert_ffn_memory",
        interpret=interpret,
    )(x2d, *ins)


def kernel(hidden_states, g1, b1, W_down, b_down, g2, b2, memory, W_k, b_k,
           W_v, b_v, g3, b3, W_up, b_up, layer_id):
    B, S, H = hidden_states.shape
    x2d = hidden_states.reshape(B * S, H)
    out = _run(x2d, g1, b1, W_down, b_down, g2, b2, memory, W_k, b_k,
               W_v, b_v, g3, b3, W_up, b_up)
    return out.reshape(B, S, H)
